# baseline (speedup 1.0000x reference)
import functools

import jax
import jax.numpy as jnp
from jax import lax
from jax.experimental import pallas as pl
from jax.experimental.pallas import tpu as pltpu
from jax.experimental.pallas import tpu_sc as plsc

N = 10000
E = 320000
D = 128

NC = 2
NS = 16
NW = NC * NS

C = 64
BLK = 32
NBLK = 5
CPW = BLK * NBLK
EPW = C * CPW
E_PAD = EPW * NW

N_ACC = 10112
RPS = N_ACC // NS
HOPS = (64,) * 9 + (56,)


def _sc_round_body(h_hbm, col_hbm, row_hbm, w_hbm, out_hbm,
                   colv, rowv, wv0, wv1, rb0, rb1, acc, gsem, wsem, ssem):
    c = lax.axis_index("c")
    s = lax.axis_index("s")
    wid = c * NS + s

    pltpu.sync_copy(col_hbm.at[wid, 0], colv)
    pltpu.sync_copy(row_hbm.at[wid, 0], rowv)

    zeros16 = jnp.zeros((16,), jnp.float32)

    def zrow(i, carry):
        for j in range(8):
            rb1[i, pl.ds(j * 16, 16)] = zeros16
        return carry

    lax.fori_loop(0, C, zrow, 0)

    pg0 = pltpu.async_copy(h_hbm.at[colv.at[0]], rb0, gsem)
    pw0 = pltpu.async_copy(w_hbm.at[wid, 0, 0], wv0, wsem)

    off = 0
    for hop in HOPS:
        pltpu.sync_copy(rb1.at[pl.ds(0, hop)], acc.at[pl.ds(s * RPS + off, hop)])
        off += hop
    plsc.subcore_barrier()

    for bi in range(NBLK):
        if bi > 0:
            pltpu.sync_copy(col_hbm.at[wid, bi], colv)
            pltpu.sync_copy(row_hbm.at[wid, bi], rowv)
            gd = {0: pltpu.async_copy(h_hbm.at[colv.at[0]], rb0, gsem)}
            wd = {0: pltpu.async_copy(w_hbm.at[wid, bi, 0], wv0, wsem)}
        else:
            gd = {0: pg0}
            wd = {0: pw0}
        sd = {}
        for j in range(BLK):
            rb = (rb0, rb1)[j % 2]
            wv = (wv0, wv1)[j % 2]
            nrb = (rb1, rb0)[j % 2]
            nwv = (wv1, wv0)[j % 2]
            gd[j].wait()
            wd[j].wait()

            def mul(i, mcarry, rb=rb, wv=wv):
                wrow = wv[i]
                for jj in range(8):
                    sl = pl.ds(jj * 16, 16)
                    rb[i, sl] = rb[i, sl] * wrow
                return mcarry

            lax.fori_loop(0, C, mul, 0)

            if j >= 1:
                sd[j - 1].wait()
            if j + 1 < BLK:
                gd[j + 1] = pltpu.async_copy(h_hbm.at[colv.at[j + 1]], nrb, gsem)
                wd[j + 1] = pltpu.async_copy(w_hbm.at[wid, bi, j + 1], nwv, wsem)
            sd[j] = pltpu.async_copy(rb, acc.at[rowv.at[j]], ssem, add=True)
        sd[BLK - 1].wait()

    plsc.subcore_barrier()

    off = 0
    for hop in HOPS:
        start = s * RPS + off
        pltpu.sync_copy(acc.at[pl.ds(start, hop)], rb0.at[pl.ds(0, hop)])
        pltpu.sync_copy(rb0.at[pl.ds(0, hop)], out_hbm.at[c, pl.ds(start, hop)])
        off += hop


@functools.partial(jax.jit, static_argnums=())
def _sc_round(h, col, row, w):
    mesh = plsc.VectorSubcoreMesh(
        core_axis_name="c", subcore_axis_name="s",
        num_cores=NC, num_subcores=NS)
    return pl.kernel(
        _sc_round_body,
        out_type=jax.ShapeDtypeStruct((NC, N_ACC, D), jnp.float32),
        mesh=mesh,
        scratch_types=[
            pltpu.VMEM((BLK, C), jnp.int32),
            pltpu.VMEM((BLK, C), jnp.int32),
            pltpu.VMEM((C, 16), jnp.float32),
            pltpu.VMEM((C, 16), jnp.float32),
            pltpu.VMEM((C, D), jnp.float32),
            pltpu.VMEM((C, D), jnp.float32),
            pltpu.VMEM_SHARED((N_ACC, D), jnp.float32),
            pltpu.SemaphoreType.DMA,
            pltpu.SemaphoreType.DMA,
            pltpu.SemaphoreType.DMA,
        ],
    )(h, col, row, w)


def _add_body(a_ref, b_ref, o_ref):
    o_ref[...] = a_ref[0] + b_ref[0]


def _combine(p):
    grid = 10
    blk = N // grid
    return pl.pallas_call(
        _add_body,
        out_shape=jax.ShapeDtypeStruct((N, D), jnp.float32),
        grid=(grid,),
        in_specs=[
            pl.BlockSpec((1, blk, D), lambda i: (0, i, 0)),
            pl.BlockSpec((1, blk, D), lambda i: (1, i, 0)),
        ],
        out_specs=pl.BlockSpec((blk, D), lambda i: (i, 0)),
    )(p, p)


def _linear_body(a_ref, b_ref, wt_ref, bias_ref, o_ref):
    h = a_ref[0] + b_ref[0]
    o_ref[...] = (
        jnp.dot(h, wt_ref[...], preferred_element_type=jnp.float32)
        + bias_ref[...]
    )


def _combine_linear(p, wt, b2):
    grid = 10
    blk = N // grid
    return pl.pallas_call(
        _linear_body,
        out_shape=jax.ShapeDtypeStruct((N, D), jnp.float32),
        grid=(grid,),
        in_specs=[
            pl.BlockSpec((1, blk, D), lambda i: (0, i, 0)),
            pl.BlockSpec((1, blk, D), lambda i: (1, i, 0)),
            pl.BlockSpec((D, D), lambda i: (0, 0)),
            pl.BlockSpec((1, D), lambda i: (0, 0)),
        ],
        out_specs=pl.BlockSpec((blk, D), lambda i: (i, 0)),
    )(p, p, wt, b2)


def kernel(x, edge_index, edge_weight, W, b):
    row = edge_index[0].astype(jnp.int32)
    col = edge_index[1].astype(jnp.int32)
    pad = E_PAD - E
    padidx = (jnp.arange(pad, dtype=jnp.int32) * 97) % N
    col_p = jnp.concatenate([col, padidx]).reshape(NW, NBLK, BLK, C)
    row_p = jnp.concatenate([row, padidx]).reshape(NW, NBLK, BLK, C)
    w_p = jnp.concatenate([edge_weight, jnp.zeros((pad,), jnp.float32)])
    wexp = jnp.broadcast_to(w_p[:, None], (E_PAD, 16)).reshape(
        NW, NBLK, BLK, C, 16)

    h = x
    for r in range(3):
        h = _combine(_sc_round(h, col_p, row_p, wexp))
    p = _sc_round(h, col_p, row_p, wexp)
    return _combine_linear(p, W.T, b.reshape(1, D))


# speedup vs baseline: 1.2276x; 1.2276x over previous
import functools

import jax
import jax.numpy as jnp
from jax import lax
from jax.experimental import pallas as pl
from jax.experimental.pallas import tpu as pltpu
from jax.experimental.pallas import tpu_sc as plsc

N = 10000
E = 320000
D = 128

NC = 2
NS = 16
NW = NC * NS

C = 64
BLK = 32
NBLK = 5
CPW = BLK * NBLK
EPW = C * CPW
E_PAD = EPW * NW

N_ACC = 10112
RPS = N_ACC // NS
HOPS = (64,) * 9 + (56,)


def _sc_round_body(h_hbm, col_hbm, row_hbm, w_hbm, out_hbm,
                   colv0, colv1, rowv0, rowv1, wv0, wv1, rb0, rb1,
                   acc, gsem, wsem, isem):
    c = lax.axis_index("c")
    s = lax.axis_index("s")
    wid = c * NS + s

    pltpu.sync_copy(col_hbm.at[wid, 0], colv0)
    pltpu.sync_copy(row_hbm.at[wid, 0], rowv0)

    zeros16 = jnp.zeros((16,), jnp.float32)

    def zrow(i, carry):
        for j in range(8):
            rb1[i, pl.ds(j * 16, 16)] = zeros16
        return carry

    lax.fori_loop(0, C, zrow, 0)

    pltpu.make_async_copy(h_hbm.at[colv0.at[0]], rb0, gsem).start()
    pltpu.make_async_copy(w_hbm.at[wid, 0, 0], wv0, wsem).start()

    off = 0
    for hop in HOPS:
        pltpu.sync_copy(rb1.at[pl.ds(0, hop)], acc.at[pl.ds(s * RPS + off, hop)])
        off += hop
    plsc.subcore_barrier()

    for bi in range(NBLK):
        cv = (colv0, colv1)[bi % 2]
        rv = (rowv0, rowv1)[bi % 2]
        ncv = (colv1, colv0)[bi % 2]
        nrv = (rowv1, rowv0)[bi % 2]
        if bi + 1 < NBLK:
            icd = pltpu.async_copy(col_hbm.at[wid, bi + 1], ncv, isem)
            ird = pltpu.async_copy(row_hbm.at[wid, bi + 1], nrv, isem)

        def inner(kk, carry, bi=bi, cv=cv, rv=rv):
            j0 = kk * 2
            for b in range(2):
                j = j0 + b
                rb = (rb0, rb1)[b]
                wv = (wv0, wv1)[b]
                nrb = (rb1, rb0)[b]
                nwv = (wv1, wv0)[b]
                pltpu.make_async_copy(h_hbm.at[cv.at[j]], rb, gsem).wait()
                pltpu.make_async_copy(w_hbm.at[wid, bi, j], wv, wsem).wait()

                @pl.when(j + 1 < BLK)
                def _():
                    pltpu.make_async_copy(
                        h_hbm.at[cv.at[j + 1]], nrb, gsem).start()
                    pltpu.make_async_copy(
                        w_hbm.at[wid, bi, j + 1], nwv, wsem).start()

                def mul(i, mcarry):
                    wrow = wv[i]
                    for jj in range(8):
                        sl = pl.ds(jj * 16, 16)
                        rb[i, sl] = rb[i, sl] * wrow
                    return mcarry

                lax.fori_loop(0, C, mul, 0)
                pltpu.sync_copy(rb, acc.at[rv.at[j]], add=True)
            return carry

        lax.fori_loop(0, BLK // 2, inner, 0)

        if bi + 1 < NBLK:
            icd.wait()
            ird.wait()
            pltpu.make_async_copy(h_hbm.at[ncv.at[0]], rb0, gsem).start()
            pltpu.make_async_copy(w_hbm.at[wid, bi + 1, 0], wv0, wsem).start()

    plsc.subcore_barrier()

    off = 0
    for hop in HOPS:
        start = s * RPS + off
        pltpu.sync_copy(acc.at[pl.ds(start, hop)], rb0.at[pl.ds(0, hop)])
        pltpu.sync_copy(rb0.at[pl.ds(0, hop)], out_hbm.at[c, pl.ds(start, hop)])
        off += hop


@functools.partial(jax.jit, static_argnums=())
def _sc_round(h, col, row, w):
    mesh = plsc.VectorSubcoreMesh(
        core_axis_name="c", subcore_axis_name="s",
        num_cores=NC, num_subcores=NS)
    return pl.kernel(
        _sc_round_body,
        out_type=jax.ShapeDtypeStruct((NC, N_ACC, D), jnp.float32),
        mesh=mesh,
        scratch_types=[
            pltpu.VMEM((BLK, C), jnp.int32),
            pltpu.VMEM((BLK, C), jnp.int32),
            pltpu.VMEM((BLK, C), jnp.int32),
            pltpu.VMEM((BLK, C), jnp.int32),
            pltpu.VMEM((C, 16), jnp.float32),
            pltpu.VMEM((C, 16), jnp.float32),
            pltpu.VMEM((C, D), jnp.float32),
            pltpu.VMEM((C, D), jnp.float32),
            pltpu.VMEM_SHARED((N_ACC, D), jnp.float32),
            pltpu.SemaphoreType.DMA,
            pltpu.SemaphoreType.DMA,
            pltpu.SemaphoreType.DMA,
        ],
    )(h, col, row, w)


def _add_body(a_ref, b_ref, o_ref):
    o_ref[...] = a_ref[0] + b_ref[0]


def _combine(p):
    grid = 10
    blk = N // grid
    return pl.pallas_call(
        _add_body,
        out_shape=jax.ShapeDtypeStruct((N, D), jnp.float32),
        grid=(grid,),
        in_specs=[
            pl.BlockSpec((1, blk, D), lambda i: (0, i, 0)),
            pl.BlockSpec((1, blk, D), lambda i: (1, i, 0)),
        ],
        out_specs=pl.BlockSpec((blk, D), lambda i: (i, 0)),
    )(p, p)


def _linear_body(a_ref, b_ref, wt_ref, bias_ref, o_ref):
    h = a_ref[0] + b_ref[0]
    o_ref[...] = (
        jnp.dot(h, wt_ref[...], preferred_element_type=jnp.float32)
        + bias_ref[...]
    )


def _combine_linear(p, wt, b2):
    grid = 10
    blk = N // grid
    return pl.pallas_call(
        _linear_body,
        out_shape=jax.ShapeDtypeStruct((N, D), jnp.float32),
        grid=(grid,),
        in_specs=[
            pl.BlockSpec((1, blk, D), lambda i: (0, i, 0)),
            pl.BlockSpec((1, blk, D), lambda i: (1, i, 0)),
            pl.BlockSpec((D, D), lambda i: (0, 0)),
            pl.BlockSpec((1, D), lambda i: (0, 0)),
        ],
        out_specs=pl.BlockSpec((blk, D), lambda i: (i, 0)),
    )(p, p, wt, b2)


def kernel(x, edge_index, edge_weight, W, b):
    row = edge_index[0].astype(jnp.int32)
    col = edge_index[1].astype(jnp.int32)
    pad = E_PAD - E
    padidx = (jnp.arange(pad, dtype=jnp.int32) * 97) % N
    col_p = jnp.concatenate([col, padidx]).reshape(NW, NBLK, BLK, C)
    row_p = jnp.concatenate([row, padidx]).reshape(NW, NBLK, BLK, C)
    w_p = jnp.concatenate([edge_weight, jnp.zeros((pad,), jnp.float32)])
    wexp = jnp.broadcast_to(w_p[:, None], (E_PAD, 16)).reshape(
        NW, NBLK, BLK, C, 16)

    h = x
    for r in range(3):
        h = _combine(_sc_round(h, col_p, row_p, wexp))
    p = _sc_round(h, col_p, row_p, wexp)
    return _combine_linear(p, W.T, b.reshape(1, D))
